# Initial kernel scaffold; baseline (speedup 1.0000x reference)
import jax
import jax.numpy as jnp
from jax.experimental import pallas as pl


def kernel(x, W_lin0, b_lin0, W_qkv, W_dw, W_g1, b_g1, W_g2, b_g2, temperature, attn1, attn2, attn3, attn4):
    raise NotImplementedError("write your pallas kernel here")


# speedup vs baseline: 1.9692x; 1.9692x over previous
import jax
import jax.numpy as jnp
from jax import lax
from jax.experimental import pallas as pl

C = 192
H = 8
HD = C // H
IMG = 224
N = IMG * IMG
R = 8
T = R * IMG
NT = IMG // R
PAD = 8
W_EXT = T + 2 * IMG + 2 * PAD
NEG = -1e30

F32 = jnp.float32
BF16 = jnp.bfloat16


def _bdot(a, b):
    return jnp.dot(a, b, preferred_element_type=F32)


def _pass1_kernel(xc_ref, xt_ref, xb_ref, wl_ref, bl_ref, wq_ref, w9_ref,
                  wg1_ref, bg1_ref, wg2_ref, bg2_ref,
                  q_ref, k_ref, v_ref, sqq_ref, sqk_ref, gs_ref):
    i = pl.program_id(0)
    zpad = jnp.zeros((C, PAD), BF16)
    xe = jnp.concatenate([zpad, xt_ref[:, T - IMG:].astype(BF16),
                          xc_ref[...].astype(BF16),
                          xb_ref[:, :IMG].astype(BF16), zpad], axis=1)
    x1 = _bdot(wl_ref[...], xe) + bl_ref[...]
    pre = _bdot(wq_ref[...], x1.astype(BF16))
    lanes = lax.broadcasted_iota(jnp.int32, (1, W_EXT), 1)
    top_ok = (lanes >= PAD + IMG) | (i > 0)
    bot_ok = (lanes < W_EXT - PAD - IMG) | (i < NT - 1)
    pad_ok = (lanes >= PAD) & (lanes < W_EXT - PAD)
    pre = pre * (top_ok & bot_ok & pad_ok).astype(F32)

    base = PAD + IMG
    col = lax.broadcasted_iota(jnp.int32, (1, T), 1) % IMG
    lm = (col != 0).astype(F32)
    rm = (col != IMG - 1).astype(F32)
    preb = pre.astype(BF16).astype(F32)
    w9 = w9_ref[...].astype(F32)

    def tap(dy, dx):
        j = (dy + 1) * 3 + (dx + 1)
        s = base + dy * IMG + dx
        t = w9[:, j:j + 1] * preb[:, s:s + T]
        if dx == -1:
            t = t * lm
        elif dx == 1:
            t = t * rm
        return t

    y = tap(-1, -1)
    for dy, dx in [(-1, 0), (-1, 1), (0, -1), (0, 0), (0, 1),
                   (1, -1), (1, 0), (1, 1)]:
        y = y + tap(dy, dx)

    q = y[:C]
    k = y[C:2 * C]
    q_ref[...] = q
    k_ref[...] = k
    v_ref[...] = y[2 * C:]

    x1c = x1[:, base:base + T]
    g1 = jnp.maximum(_bdot(wg1_ref[...], x1c.astype(BF16)) + bg1_ref[...], 0.0)
    g2 = jax.nn.sigmoid(
        jnp.sum(wg2_ref[...].astype(F32) * g1.astype(BF16).astype(F32),
                axis=0, keepdims=True) + bg2_ref[...])

    @pl.when(i == 0)
    def _init():
        sqq_ref[...] = jnp.zeros_like(sqq_ref)
        sqk_ref[...] = jnp.zeros_like(sqk_ref)
        gs_ref[...] = jnp.zeros_like(gs_ref)

    sqq_ref[...] += jnp.sum(q * q, axis=1, keepdims=True)
    sqk_ref[...] += jnp.sum(k * k, axis=1, keepdims=True)
    gs_ref[...] += jnp.sum(g2, keepdims=True)


def _gram_kernel(q_ref, k_ref, sqq_ref, sqk_ref, g_ref):
    i = pl.program_id(0)
    nq = jnp.maximum(jnp.sqrt(sqq_ref[...]), 1e-12)
    nk = jnp.maximum(jnp.sqrt(sqk_ref[...]), 1e-12)
    qn = (q_ref[...] / nq).astype(BF16)
    kn = (k_ref[...] / nk).astype(BF16)

    @pl.when(i == 0)
    def _init():
        g_ref[...] = jnp.zeros_like(g_ref)

    g_ref[...] += lax.dot_general(qn, kn, (((1,), (1,)), ((), ())),
                                  preferred_element_type=F32)


def _mask_kernel(g_ref, gs_ref, tv_ref, a_ref):
    attn = g_ref[...] * tv_ref[...]
    blocks = [attn[h * HD:(h + 1) * HD, h * HD:(h + 1) * HD] for h in range(H)]
    b = jnp.concatenate(blocks, axis=0)
    dkf = jnp.clip(jnp.floor(HD * gs_ref[0, 0] / N), 1.0, float(HD))
    bd = b[:, :, None]
    be = b[:, None, :]
    ie = lax.broadcasted_iota(jnp.int32, (C, HD, HD), 2)
    idx = lax.broadcasted_iota(jnp.int32, (C, HD, HD), 1)
    gt = (be > bd).astype(F32)
    eq = ((be == bd) & (ie < idx)).astype(F32)
    rank = jnp.sum(gt + eq, axis=2)
    keep = rank < dkf
    keepf = keep.astype(F32)
    bm = jnp.where(keep, b, NEG)
    m = jnp.max(bm, axis=1, keepdims=True)
    e = jnp.exp(bm - m) * keepf
    s = jnp.sum(e, axis=1, keepdims=True)
    a = e / s
    at = jnp.concatenate([a] * H, axis=1)
    ic = lax.broadcasted_iota(jnp.int32, (C, C), 0) // HD
    jc = lax.broadcasted_iota(jnp.int32, (C, C), 1) // HD
    a_ref[...] = jnp.where(ic == jc, at, 0.0)


def _out_kernel(a_ref, v_ref, a1_ref, a2_ref, a3_ref, a4_ref, o_ref):
    o = _bdot(a_ref[...].astype(BF16), v_ref[...].astype(BF16))
    o_ref[...] = (o * a1_ref[0, 0] + o * a2_ref[0, 0]
                  + o * a3_ref[0, 0] + o * a4_ref[0, 0])


def kernel(x, W_lin0, b_lin0, W_qkv, W_dw, W_g1, b_g1, W_g2, b_g2,
           temperature, attn1, attn2, attn3, attn4):
    x2 = x.reshape(C, N)
    w9 = W_dw.reshape(3 * C, 9)
    wl = W_lin0.astype(BF16)
    wq = W_qkv.astype(BF16)
    wg1 = W_g1.astype(BF16)
    wg2 = W_g2.reshape(C // 2, 1).astype(BF16)
    blin = b_lin0.reshape(C, 1)
    bg1 = b_g1.reshape(C // 2, 1)
    bg2 = b_g2.reshape(1, 1)
    tv = jnp.repeat(temperature.reshape(H, 1), HD, axis=0)
    sc = lambda a: a.reshape(1, 1)

    full = lambda s: pl.BlockSpec(s, lambda i: (0, 0))
    tile = pl.BlockSpec((C, T), lambda i: (0, i))
    q, k, v, sqq, sqk, gs = pl.pallas_call(
        _pass1_kernel,
        grid=(NT,),
        in_specs=[
            tile,
            pl.BlockSpec((C, T), lambda i: (0, jnp.maximum(i - 1, 0))),
            pl.BlockSpec((C, T), lambda i: (0, jnp.minimum(i + 1, NT - 1))),
            full((C, C)),
            full((C, 1)),
            full((3 * C, C)),
            full((3 * C, 9)),
            full((C // 2, C)),
            full((C // 2, 1)),
            full((C // 2, 1)),
            full((1, 1)),
        ],
        out_specs=[tile, tile, tile, full((C, 1)), full((C, 1)), full((1, 1))],
        out_shape=[
            jax.ShapeDtypeStruct((C, N), F32),
            jax.ShapeDtypeStruct((C, N), F32),
            jax.ShapeDtypeStruct((C, N), F32),
            jax.ShapeDtypeStruct((C, 1), F32),
            jax.ShapeDtypeStruct((C, 1), F32),
            jax.ShapeDtypeStruct((1, 1), F32),
        ],
    )(x2, x2, x2, wl, blin, wq, w9, wg1, bg1, wg2, bg2)

    g = pl.pallas_call(
        _gram_kernel,
        grid=(NT,),
        in_specs=[tile, tile, full((C, 1)), full((C, 1))],
        out_specs=full((C, C)),
        out_shape=jax.ShapeDtypeStruct((C, C), F32),
    )(q, k, sqq, sqk)

    a_bd = pl.pallas_call(
        _mask_kernel,
        out_shape=jax.ShapeDtypeStruct((C, C), F32),
    )(g, gs, tv)

    o = pl.pallas_call(
        _out_kernel,
        grid=(NT,),
        in_specs=[full((C, C)), tile, full((1, 1)), full((1, 1)),
                  full((1, 1)), full((1, 1))],
        out_specs=tile,
        out_shape=jax.ShapeDtypeStruct((C, N), F32),
    )(a_bd, v, sc(attn1), sc(attn2), sc(attn3), sc(attn4))

    return o.reshape(1, C, IMG, IMG)
